# baseline (speedup 1.0000x reference)
import functools

import jax
import jax.numpy as jnp
from jax import lax
from jax.experimental import pallas as pl
from jax.experimental.pallas import tpu as pltpu
from jax.experimental.pallas import tpu_sc as plsc

NC = 2
NS = 16
NW = NC * NS
CHUNK = 128
NB = 4


def _sc_aggregate(n, n_pad, d, chunks_per_worker):
  mesh = plsc.VectorSubcoreMesh(core_axis_name="c", subcore_axis_name="s")
  rows_per_tile = n_pad // NS
  n_blocks = chunks_per_worker // NB

  @functools.partial(
      pl.kernel,
      out_type=[
          jax.ShapeDtypeStruct((NC, n_pad, d), jnp.float32),
          jax.ShapeDtypeStruct((NC, n_pad), jnp.float32),
      ],
      mesh=mesh,
      scratch_types=[
          pltpu.VMEM_SHARED((n_pad, d), jnp.float32),
          pltpu.VMEM_SHARED((n_pad,), jnp.float32),
          pltpu.VMEM((NB, CHUNK), jnp.int32),
          pltpu.VMEM((NB, CHUNK), jnp.int32),
          pltpu.VMEM((NB, CHUNK), jnp.float32),
          pltpu.VMEM((CHUNK, d), jnp.float32),
          pltpu.VMEM((n_pad // NS,), jnp.float32),
          pltpu.SemaphoreType.DMA,
      ],
  )
  def k(x_hbm, src_hbm, dst_hbm, w_hbm, out_hbm, deg_hbm,
        acc, dacc, sidx, didx, wv, rows, zdeg, gsem):
    cid = lax.axis_index("c")
    sid = lax.axis_index("s")
    wid = sid * NC + cid
    zvec = jnp.zeros((16,), jnp.float32)
    n_sub = rows_per_tile // CHUNK

    def zrow(i, _):
      for j in range(d // 16):
        rows[i, pl.ds(j * 16, 16)] = zvec
      return 0
    lax.fori_loop(0, CHUNK, zrow, 0)

    def zdrow(i, _):
      zdeg[pl.ds(i * 16, 16)] = zvec
      return 0
    lax.fori_loop(0, rows_per_tile // 16, zdrow, 0)

    row0 = sid * rows_per_tile

    def zcopy(t, _):
      pltpu.sync_copy(rows, acc.at[pl.ds(row0 + t * CHUNK, CHUNK), :])
      return 0
    lax.fori_loop(0, n_sub, zcopy, 0)
    pltpu.sync_copy(zdeg, dacc.at[pl.ds(row0, rows_per_tile)])
    plsc.subcore_barrier()

    def weight_deg(l):
      def group_body(g, _):
        w16 = wv[l, pl.ds(g * 16, 16)]
        for i in range(16):
          ws = w16[i]
          e = g * 16 + i
          for j in range(d // 16):
            sl = pl.ds(j * 16, 16)
            rows[e, sl] = rows[e, sl] * ws
        return 0
      lax.fori_loop(0, CHUNK // 16, group_body, 0)

    def block_body(blk, _):
      c0 = blk * NB
      pltpu.sync_copy(src_hbm.at[wid, pl.ds(c0, NB), :], sidx)
      pltpu.sync_copy(dst_hbm.at[wid, pl.ds(c0, NB), :], didx)
      pltpu.sync_copy(w_hbm.at[wid, pl.ds(c0, NB), :], wv)
      for l in range(NB):
        pltpu.async_copy(x_hbm.at[sidx.at[l]], rows, gsem).wait()
        weight_deg(l)
        pltpu.sync_copy(rows, acc.at[didx.at[l]], add=True)
        pltpu.sync_copy(wv.at[l], dacc.at[didx.at[l]], add=True)
      return 0

    lax.fori_loop(0, n_blocks, block_body, 0)
    plsc.subcore_barrier()

    pltpu.sync_copy(acc.at[pl.ds(row0, rows_per_tile), :],
                    out_hbm.at[cid, pl.ds(row0, rows_per_tile), :])
    pltpu.sync_copy(dacc.at[pl.ds(row0, rows_per_tile)],
                    deg_hbm.at[cid, pl.ds(row0, rows_per_tile)])

  return k


def _tc_mlp(p_ref, dg_ref, x_ref, eps_ref, w1_ref, b1_ref, w2_ref, b2_ref,
            o_ref):
  p = p_ref[0] + p_ref[1]
  dg = dg_ref[0] + dg_ref[1]
  agg = p / jnp.maximum(dg, 1e-8)
  h = (1.0 + eps_ref[0, 0]) * x_ref[...] + agg
  h = jnp.dot(h, w1_ref[...], preferred_element_type=jnp.float32)
  h = jnp.maximum(h + b1_ref[...], 0.0)
  h = jnp.dot(h, w2_ref[...], preferred_element_type=jnp.float32)
  o_ref[...] = h + b2_ref[...]


def kernel(x, edge_index, edge_weight, eps, W1, b1, W2, b2):
  n, d = x.shape
  e = edge_index.shape[1]
  blk = 400
  n_pad = ((n + NS * CHUNK - 1) // (NS * CHUNK)) * (NS * CHUNK)
  step = NW * NB * CHUNK
  e_pad = ((e + step - 1) // step) * step
  chunks_per_worker = e_pad // (NW * CHUNK)

  src = jnp.pad(edge_index[0], (0, e_pad - e)).reshape(NW, -1, CHUNK)
  dst = jnp.pad(edge_index[1], (0, e_pad - e)).reshape(NW, -1, CHUNK)
  w = jnp.pad(edge_weight, (0, e_pad - e)).reshape(NW, -1, CHUNK)

  out_p, deg_p = _sc_aggregate(n, n_pad, d, chunks_per_worker)(x, src, dst, w)

  deg_p = deg_p.reshape(NC, n_pad, 1)
  grid = (n // blk,)
  return pl.pallas_call(
      _tc_mlp,
      grid=grid,
      in_specs=[
          pl.BlockSpec((NC, blk, d), lambda i: (0, i, 0)),
          pl.BlockSpec((NC, blk, 1), lambda i: (0, i, 0)),
          pl.BlockSpec((blk, d), lambda i: (i, 0)),
          pl.BlockSpec((1, 1), lambda i: (0, 0)),
          pl.BlockSpec((d, d), lambda i: (0, 0)),
          pl.BlockSpec((1, d), lambda i: (0, 0)),
          pl.BlockSpec((d, d), lambda i: (0, 0)),
          pl.BlockSpec((1, d), lambda i: (0, 0)),
      ],
      out_specs=pl.BlockSpec((blk, d), lambda i: (i, 0)),
      out_shape=jax.ShapeDtypeStruct((n, d), jnp.float32),
  )(out_p, deg_p, x, eps.reshape(1, 1), W1, b1.reshape(1, d), W2,
    b2.reshape(1, d))


# speedup vs baseline: 1.1021x; 1.1021x over previous
import functools

import jax
import jax.numpy as jnp
from jax import lax
from jax.experimental import pallas as pl
from jax.experimental.pallas import tpu as pltpu
from jax.experimental.pallas import tpu_sc as plsc

NC = 2
NS = 16
NW = NC * NS
CHUNK = 128


def _sc_aggregate(n_pad, d, cpw0, cpw1):
  mesh = plsc.VectorSubcoreMesh(core_axis_name="c", subcore_axis_name="s")
  rows_per_tile = n_pad // NS

  @functools.partial(
      pl.kernel,
      out_type=[
          jax.ShapeDtypeStruct((NC, n_pad, d), jnp.float32),
          jax.ShapeDtypeStruct((NC, n_pad), jnp.float32),
      ],
      mesh=mesh,
      scratch_types=[
          pltpu.VMEM_SHARED((n_pad, d), jnp.float32),
          pltpu.VMEM_SHARED((n_pad,), jnp.float32),
          pltpu.VMEM((CHUNK,), jnp.int32),
          pltpu.VMEM((CHUNK,), jnp.int32),
          pltpu.VMEM((CHUNK,), jnp.float32),
          pltpu.VMEM((CHUNK, d), jnp.float32),
          pltpu.VMEM((rows_per_tile,), jnp.float32),
          pltpu.SemaphoreType.DMA,
      ],
  )
  def k(x_hbm, src_hbm, dst_hbm, w_hbm, out_hbm, deg_hbm,
        acc, dacc, sidx, didx, wv, rows, zdeg, sem):
    cid = lax.axis_index("c")
    sid = lax.axis_index("s")
    zvec = jnp.zeros((16,), jnp.float32)
    n_sub = rows_per_tile // CHUNK
    chunk0 = jnp.where(cid == 0, sid * cpw0, NS * cpw0 + sid * cpw1)
    n_chunks = jnp.where(cid == 0, cpw0, cpw1)

    def zrow(i, _):
      for j in range(d // 16):
        rows[i, pl.ds(j * 16, 16)] = zvec
      return 0
    lax.fori_loop(0, CHUNK, zrow, 0)

    def zdrow(i, _):
      zdeg[pl.ds(i * 16, 16)] = zvec
      return 0
    lax.fori_loop(0, rows_per_tile // 16, zdrow, 0)

    row0 = sid * rows_per_tile

    def zcopy(t, _):
      pltpu.sync_copy(rows, acc.at[pl.ds(row0 + t * CHUNK, CHUNK), :])
      return 0
    lax.fori_loop(0, n_sub, zcopy, 0)
    pltpu.sync_copy(zdeg, dacc.at[pl.ds(row0, rows_per_tile)])
    plsc.subcore_barrier()

    def chunk_body(c, _):
      base = (chunk0 + c) * CHUNK
      pltpu.sync_copy(src_hbm.at[pl.ds(base, CHUNK)], sidx)
      pltpu.sync_copy(dst_hbm.at[pl.ds(base, CHUNK)], didx)
      pltpu.sync_copy(w_hbm.at[pl.ds(base, CHUNK)], wv)
      pltpu.async_copy(x_hbm.at[sidx], rows, sem).wait()

      def group_body(g, _):
        w16 = wv[pl.ds(g * 16, 16)]
        for i in range(16):
          ws = w16[i]
          e = g * 16 + i
          for j in range(d // 16):
            sl = pl.ds(j * 16, 16)
            rows[e, sl] = rows[e, sl] * ws
        return 0
      lax.fori_loop(0, CHUNK // 16, group_body, 0)

      pltpu.sync_copy(rows, acc.at[didx], add=True)
      pltpu.sync_copy(wv, dacc.at[didx], add=True)
      return 0
    lax.fori_loop(0, n_chunks, chunk_body, 0)

    plsc.subcore_barrier()

    def ocopy(t, _):
      r = row0 + t * CHUNK
      pltpu.sync_copy(acc.at[pl.ds(r, CHUNK), :], rows)
      pltpu.sync_copy(rows, out_hbm.at[cid, pl.ds(r, CHUNK), :])
      return 0
    lax.fori_loop(0, n_sub, ocopy, 0)
    pltpu.sync_copy(dacc.at[pl.ds(row0, rows_per_tile)], zdeg)
    pltpu.sync_copy(zdeg, deg_hbm.at[cid, pl.ds(row0, rows_per_tile)])

  return k


def _tc_mlp(p_ref, dg_ref, x_ref, eps_ref, w1_ref, b1_ref, w2_ref, b2_ref,
            o_ref):
  p = p_ref[0] + p_ref[1]
  dg = dg_ref[0] + dg_ref[1]
  agg = p / jnp.maximum(dg, 1e-8)
  h = (1.0 + eps_ref[0, 0]) * x_ref[...] + agg
  h = jnp.dot(h, w1_ref[...], preferred_element_type=jnp.float32)
  h = jnp.maximum(h + b1_ref[...], 0.0)
  h = jnp.dot(h, w2_ref[...], preferred_element_type=jnp.float32)
  o_ref[...] = h + b2_ref[...]


def kernel(x, edge_index, edge_weight, eps, W1, b1, W2, b2):
  n, d = x.shape
  e = edge_index.shape[1]
  blk = 400
  n_pad = ((n + NS * CHUNK - 1) // (NS * CHUNK)) * (NS * CHUNK)
  step = NW * CHUNK
  e_pad = ((e + step - 1) // step) * step
  tot = e_pad // (NS * CHUNK)
  cpw0 = max(1, 2 * tot // 5)
  cpw1 = tot - cpw0

  src = jnp.pad(edge_index[0], (0, e_pad - e))
  dst = jnp.pad(edge_index[1], (0, e_pad - e))
  w = jnp.pad(edge_weight, (0, e_pad - e))

  out_p, deg_p = _sc_aggregate(n_pad, d, cpw0, cpw1)(x, src, dst, w)

  deg_p = deg_p.reshape(NC, n_pad, 1)
  grid = (n // blk,)
  return pl.pallas_call(
      _tc_mlp,
      grid=grid,
      in_specs=[
          pl.BlockSpec((NC, blk, d), lambda i: (0, i, 0)),
          pl.BlockSpec((NC, blk, 1), lambda i: (0, i, 0)),
          pl.BlockSpec((blk, d), lambda i: (i, 0)),
          pl.BlockSpec((1, 1), lambda i: (0, 0)),
          pl.BlockSpec((d, d), lambda i: (0, 0)),
          pl.BlockSpec((1, d), lambda i: (0, 0)),
          pl.BlockSpec((d, d), lambda i: (0, 0)),
          pl.BlockSpec((1, d), lambda i: (0, 0)),
      ],
      out_specs=pl.BlockSpec((blk, d), lambda i: (i, 0)),
      out_shape=jax.ShapeDtypeStruct((n, d), jnp.float32),
  )(out_p, deg_p, x, eps.reshape(1, 1), W1, b1.reshape(1, d), W2,
    b2.reshape(1, d))


# speedup vs baseline: 1.3840x; 1.2558x over previous
import functools

import jax
import jax.numpy as jnp
from jax import lax
from jax.experimental import pallas as pl
from jax.experimental.pallas import tpu as pltpu
from jax.experimental.pallas import tpu_sc as plsc

NC = 2
NS = 16
NW = NC * NS
CHUNK = 128


def _sc_aggregate(n_pad, d, cpw0, cpw1):
  mesh = plsc.VectorSubcoreMesh(core_axis_name="c", subcore_axis_name="s")
  rows_per_tile = n_pad // NS

  @functools.partial(
      pl.kernel,
      out_type=[
          jax.ShapeDtypeStruct((NC, n_pad, d), jnp.float32),
          jax.ShapeDtypeStruct((NC, n_pad), jnp.float32),
      ],
      mesh=mesh,
      scratch_types=[
          pltpu.VMEM_SHARED((n_pad, d), jnp.float32),
          pltpu.VMEM_SHARED((n_pad,), jnp.float32),
          pltpu.VMEM((CHUNK,), jnp.int32),
          pltpu.VMEM((CHUNK,), jnp.int32),
          pltpu.VMEM((CHUNK,), jnp.float32),
          pltpu.VMEM((CHUNK, d), jnp.float32),
          pltpu.VMEM((rows_per_tile,), jnp.float32),
          pltpu.SemaphoreType.DMA,
      ],
  )
  def k(x_hbm, src_hbm, dst_hbm, w_hbm, out_hbm, deg_hbm,
        acc, dacc, sidx, didx, wv, rows, zdeg, sem):
    cid = lax.axis_index("c")
    sid = lax.axis_index("s")
    zvec = jnp.zeros((16,), jnp.float32)
    n_sub = rows_per_tile // CHUNK
    chunk0 = jnp.where(cid == 0, sid * cpw0, NS * cpw0 + sid * cpw1)
    n_chunks = jnp.where(cid == 0, cpw0, cpw1)

    def zrow(i, _):
      for j in range(d // 16):
        rows[i, pl.ds(j * 16, 16)] = zvec
      return 0
    lax.fori_loop(0, CHUNK, zrow, 0)

    def zdrow(i, _):
      zdeg[pl.ds(i * 16, 16)] = zvec
      return 0
    lax.fori_loop(0, rows_per_tile // 16, zdrow, 0)

    row0 = sid * rows_per_tile

    def zcopy(t, _):
      pltpu.sync_copy(rows, acc.at[pl.ds(row0 + t * CHUNK, CHUNK), :])
      return 0
    lax.fori_loop(0, n_sub, zcopy, 0)
    pltpu.sync_copy(zdeg, dacc.at[pl.ds(row0, rows_per_tile)])
    plsc.subcore_barrier()

    def chunk_body(c, _):
      base = (chunk0 + c) * CHUNK
      pltpu.sync_copy(src_hbm.at[pl.ds(base, CHUNK)], sidx)
      pltpu.sync_copy(dst_hbm.at[pl.ds(base, CHUNK)], didx)
      pltpu.sync_copy(w_hbm.at[pl.ds(base, CHUNK)], wv)
      pltpu.async_copy(x_hbm.at[sidx], rows, sem).wait()

      def group_body(g, _):
        w16 = wv[pl.ds(g * 16, 16)]
        for i in range(16):
          ws = w16[i]
          e = g * 16 + i
          for j in range(d // 16):
            sl = pl.ds(j * 16, 16)
            rows[e, sl] = rows[e, sl] * ws
        return 0
      lax.fori_loop(0, CHUNK // 16, group_body, 0)

      pltpu.sync_copy(rows, acc.at[didx], add=True)
      pltpu.sync_copy(wv, dacc.at[didx], add=True)
      return 0
    lax.fori_loop(0, n_chunks, chunk_body, 0)

    plsc.subcore_barrier()

    def ocopy(t, _):
      r = row0 + t * CHUNK
      pltpu.sync_copy(acc.at[pl.ds(r, CHUNK), :], rows)
      pltpu.sync_copy(rows, out_hbm.at[cid, pl.ds(r, CHUNK), :])
      return 0
    lax.fori_loop(0, n_sub, ocopy, 0)
    pltpu.sync_copy(dacc.at[pl.ds(row0, rows_per_tile)], zdeg)
    pltpu.sync_copy(zdeg, deg_hbm.at[cid, pl.ds(row0, rows_per_tile)])

  return k


def _tc_mlp(p_ref, dg_ref, x_ref, eps_ref, w1_ref, b1_ref, w2_ref, b2_ref,
            o_ref):
  p = p_ref[0] + p_ref[1]
  dg = dg_ref[0] + dg_ref[1]
  agg = p / jnp.maximum(dg, 1e-8)
  h = (1.0 + eps_ref[0, 0]) * x_ref[...] + agg
  h = jnp.dot(h, w1_ref[...], preferred_element_type=jnp.float32)
  h = jnp.maximum(h + b1_ref[...], 0.0)
  h = jnp.dot(h, w2_ref[...], preferred_element_type=jnp.float32)
  o_ref[...] = h + b2_ref[...]


def kernel(x, edge_index, edge_weight, eps, W1, b1, W2, b2):
  n, d = x.shape
  e = edge_index.shape[1]
  blk = 400
  n_pad = ((n + NS * CHUNK - 1) // (NS * CHUNK)) * (NS * CHUNK)
  step = NW * CHUNK
  e_pad = ((e + step - 1) // step) * step
  tot = e_pad // (NS * CHUNK)
  cpw1 = max(1, 2 * tot // 5)
  cpw0 = tot - cpw1

  src = jnp.pad(edge_index[0], (0, e_pad - e))
  dst = jnp.pad(edge_index[1], (0, e_pad - e))
  w = jnp.pad(edge_weight, (0, e_pad - e))

  out_p, deg_p = _sc_aggregate(n_pad, d, cpw0, cpw1)(x, src, dst, w)

  deg_p = deg_p.reshape(NC, n_pad, 1)
  grid = (n // blk,)
  return pl.pallas_call(
      _tc_mlp,
      grid=grid,
      in_specs=[
          pl.BlockSpec((NC, blk, d), lambda i: (0, i, 0)),
          pl.BlockSpec((NC, blk, 1), lambda i: (0, i, 0)),
          pl.BlockSpec((blk, d), lambda i: (i, 0)),
          pl.BlockSpec((1, 1), lambda i: (0, 0)),
          pl.BlockSpec((d, d), lambda i: (0, 0)),
          pl.BlockSpec((1, d), lambda i: (0, 0)),
          pl.BlockSpec((d, d), lambda i: (0, 0)),
          pl.BlockSpec((1, d), lambda i: (0, 0)),
      ],
      out_specs=pl.BlockSpec((blk, d), lambda i: (i, 0)),
      out_shape=jax.ShapeDtypeStruct((n, d), jnp.float32),
  )(out_p, deg_p, x, eps.reshape(1, 1), W1, b1.reshape(1, d), W2,
    b2.reshape(1, d))
